# baseline (speedup 1.0000x reference)
import functools

import jax
import jax.numpy as jnp
from jax import lax
from jax.experimental import pallas as pl
from jax.experimental.pallas import tpu as pltpu
from jax.experimental.pallas import tpu_sc as plsc

_ROWS = 4096
_CHUNK = 16

_SC_ROWS = 4096
_NC, _NS, _L = 2, 16, 16
_NW = _NC * _NS
_SC_CHUNK = 8192


def _invmae_body(p_ref, t_ref, out_ref, vacc_ref, cacc_ref):
    i = pl.program_id(0)

    @pl.when(i == 0)
    def _init():
        vacc_ref[...] = jnp.zeros_like(vacc_ref)
        cacc_ref[...] = jnp.zeros_like(cacc_ref)

    def body(k, carry):
        aerr, acnt = carry
        sl = pl.ds(k * _CHUNK, _CHUNK)
        p = p_ref[sl, :]
        t = t_ref[sl, :]
        mask = t > 0.0
        err = jnp.where(mask, jnp.abs(p - t) / (p * t), 0.0)
        cnt = jnp.where(mask, 1.0, 0.0)
        aerr += jnp.sum(err.reshape(_CHUNK // 8, 8, 512), axis=0)
        acnt += jnp.sum(cnt.reshape(_CHUNK // 8, 8, 512), axis=0)
        return aerr, acnt

    zero = jnp.zeros((8, 512), jnp.float32)
    aerr, acnt = jax.lax.fori_loop(
        0, _ROWS // _CHUNK, body, (zero, zero), unroll=4
    )
    vacc_ref[...] += aerr
    cacc_ref[...] += acnt

    @pl.when(i == pl.num_programs(0) - 1)
    def _fin():
        out_ref[0] = jnp.sum(vacc_ref[...])
        out_ref[1] = jnp.sum(cacc_ref[...])


def _tc_partial(p, t):
    rows = p.shape[0]
    return pl.pallas_call(
        _invmae_body,
        grid=(rows // _ROWS,),
        in_specs=[
            pl.BlockSpec((_ROWS, 512), lambda i: (i, 0)),
            pl.BlockSpec((_ROWS, 512), lambda i: (i, 0)),
        ],
        out_specs=pl.BlockSpec(memory_space=pltpu.SMEM),
        out_shape=jax.ShapeDtypeStruct((2,), jnp.float32),
        scratch_shapes=[
            pltpu.VMEM((8, 512), jnp.float32),
            pltpu.VMEM((8, 512), jnp.float32),
        ],
    )(p, t)


def _sc_make(sc_n):
    per_w = sc_n // _NW
    n_chunks = per_w // _SC_CHUNK
    mesh = plsc.VectorSubcoreMesh(core_axis_name="c", subcore_axis_name="s")

    @functools.partial(
        pl.kernel,
        mesh=mesh,
        out_type=jax.ShapeDtypeStruct((_NW, 2, _L), jnp.float32),
        scratch_types=[
            pltpu.VMEM((_SC_CHUNK,), jnp.float32),
            pltpu.VMEM((_SC_CHUNK,), jnp.float32),
            pltpu.VMEM((2, _L), jnp.float32),
        ],
    )
    def sc_kernel(p_hbm, t_hbm, out_hbm, p_v, t_v, res_v):
        wid = lax.axis_index("s") * _NC + lax.axis_index("c")
        base = wid * per_w

        def chunk_body(g, carry):
            aerr, acnt = carry
            off = base + g * _SC_CHUNK
            pltpu.sync_copy(p_hbm.at[pl.ds(off, _SC_CHUNK)], p_v)
            pltpu.sync_copy(t_hbm.at[pl.ds(off, _SC_CHUNK)], t_v)

            def inner(j, c2):
                a2, n2 = c2
                sl = pl.ds(j * _L, _L)
                p = p_v[sl]
                t = t_v[sl]
                mask = t > 0.0
                err = jnp.where(mask, jnp.abs(p - t) / (p * t), 0.0)
                one = jnp.where(mask, 1.0, 0.0)
                return a2 + err, n2 + one

            return lax.fori_loop(0, _SC_CHUNK // _L, inner, (aerr, acnt))

        zero = jnp.zeros((_L,), jnp.float32)
        aerr, acnt = lax.fori_loop(0, n_chunks, chunk_body, (zero, zero))
        res_v[0, :] = aerr
        res_v[1, :] = acnt
        pltpu.sync_copy(res_v, out_hbm.at[wid])

    return sc_kernel


def kernel(pred, target):
    n = pred.size
    rows = n // 512
    tc_rows = rows - _SC_ROWS
    p2 = pred.reshape(rows, 512)
    t2 = target.reshape(rows, 512)

    tc_out = _tc_partial(p2[:tc_rows], t2[:tc_rows])

    sc_n = _SC_ROWS * 512
    sc_parts = _sc_make(sc_n)(
        p2[tc_rows:].reshape(sc_n), t2[tc_rows:].reshape(sc_n)
    )

    s = tc_out[0] + jnp.sum(sc_parts[:, 0, :])
    c = tc_out[1] + jnp.sum(sc_parts[:, 1, :])
    loss = s / jnp.maximum(c, 1.0)
    return jnp.where(c < 10.0, jnp.float32(-1.0), loss)


# speedup vs baseline: 1.0337x; 1.0337x over previous
import functools

import jax
import jax.numpy as jnp
from jax import lax
from jax.experimental import pallas as pl
from jax.experimental.pallas import tpu as pltpu
from jax.experimental.pallas import tpu_sc as plsc

_ROWS = 4096
_CHUNK = 16

_SC_ROWS = 2048
_NC, _NS, _L = 2, 16, 16
_NW = _NC * _NS
_SC_CHUNK = 8192


def _invmae_body(p_ref, t_ref, out_ref, vacc_ref, cacc_ref):
    i = pl.program_id(0)

    @pl.when(i == 0)
    def _init():
        vacc_ref[...] = jnp.zeros_like(vacc_ref)
        cacc_ref[...] = jnp.zeros_like(cacc_ref)

    def body(k, carry):
        aerr, acnt = carry
        sl = pl.ds(k * _CHUNK, _CHUNK)
        p = p_ref[sl, :]
        t = t_ref[sl, :]
        mask = t > 0.0
        err = jnp.where(mask, jnp.abs(p - t) / (p * t), 0.0)
        cnt = jnp.where(mask, 1.0, 0.0)
        aerr += jnp.sum(err.reshape(_CHUNK // 8, 8, 512), axis=0)
        acnt += jnp.sum(cnt.reshape(_CHUNK // 8, 8, 512), axis=0)
        return aerr, acnt

    zero = jnp.zeros((8, 512), jnp.float32)
    aerr, acnt = jax.lax.fori_loop(
        0, _ROWS // _CHUNK, body, (zero, zero), unroll=4
    )
    vacc_ref[...] += aerr
    cacc_ref[...] += acnt

    @pl.when(i == pl.num_programs(0) - 1)
    def _fin():
        out_ref[0] = jnp.sum(vacc_ref[...])
        out_ref[1] = jnp.sum(cacc_ref[...])


def _tc_partial(p, t, tc_rows):
    return pl.pallas_call(
        _invmae_body,
        grid=(tc_rows // _ROWS,),
        in_specs=[
            pl.BlockSpec((_ROWS, 512), lambda i: (i, 0)),
            pl.BlockSpec((_ROWS, 512), lambda i: (i, 0)),
        ],
        out_specs=pl.BlockSpec(memory_space=pltpu.SMEM),
        out_shape=jax.ShapeDtypeStruct((2,), jnp.float32),
        scratch_shapes=[
            pltpu.VMEM((8, 512), jnp.float32),
            pltpu.VMEM((8, 512), jnp.float32),
        ],
    )(p, t)


def _sc_make(sc_n, sc_base):
    per_w = sc_n // _NW
    n_chunks = per_w // _SC_CHUNK
    mesh = plsc.VectorSubcoreMesh(core_axis_name="c", subcore_axis_name="s")

    @functools.partial(
        pl.kernel,
        mesh=mesh,
        out_type=jax.ShapeDtypeStruct((_NW, 2, _L), jnp.float32),
        scratch_types=[
            pltpu.VMEM((_SC_CHUNK,), jnp.float32),
            pltpu.VMEM((_SC_CHUNK,), jnp.float32),
            pltpu.VMEM((2, _L), jnp.float32),
        ],
    )
    def sc_kernel(p_hbm, t_hbm, out_hbm, p_v, t_v, res_v):
        wid = lax.axis_index("s") * _NC + lax.axis_index("c")
        base = sc_base + wid * per_w

        def chunk_body(g, carry):
            aerr, acnt = carry
            off = base + g * _SC_CHUNK
            pltpu.sync_copy(p_hbm.at[pl.ds(off, _SC_CHUNK)], p_v)
            pltpu.sync_copy(t_hbm.at[pl.ds(off, _SC_CHUNK)], t_v)

            def inner(j, c2):
                a2, n2 = c2
                sl = pl.ds(j * _L, _L)
                p = p_v[sl]
                t = t_v[sl]
                mask = t > 0.0
                err = jnp.where(mask, jnp.abs(p - t) / (p * t), 0.0)
                one = jnp.where(mask, 1.0, 0.0)
                return a2 + err, n2 + one

            return lax.fori_loop(
                0, _SC_CHUNK // _L, inner, (aerr, acnt), unroll=8
            )

        zero = jnp.zeros((_L,), jnp.float32)
        aerr, acnt = lax.fori_loop(0, n_chunks, chunk_body, (zero, zero))
        res_v[0, :] = aerr
        res_v[1, :] = acnt
        pltpu.sync_copy(res_v, out_hbm.at[wid])

    return sc_kernel


def kernel(pred, target):
    n = pred.size
    rows = n // 512
    tc_rows = rows - _SC_ROWS
    p2 = pred.reshape(rows, 512)
    t2 = target.reshape(rows, 512)

    tc_out = _tc_partial(p2, t2, tc_rows)

    sc_n = _SC_ROWS * 512
    sc_parts = _sc_make(sc_n, tc_rows * 512)(p2.reshape(n), t2.reshape(n))

    s = tc_out[0] + jnp.sum(sc_parts[:, 0, :])
    c = tc_out[1] + jnp.sum(sc_parts[:, 1, :])
    loss = s / jnp.maximum(c, 1.0)
    return jnp.where(c < 10.0, jnp.float32(-1.0), loss)


# speedup vs baseline: 2.6958x; 2.6079x over previous
import functools

import jax
import jax.numpy as jnp
from jax import lax
from jax.experimental import pallas as pl
from jax.experimental.pallas import tpu as pltpu
from jax.experimental.pallas import tpu_sc as plsc

_ROWS = 4096
_CHUNK = 16

_SC_ROWS = 2048
_NC, _NS, _L = 2, 16, 16
_NW = _NC * _NS
_SC_CROWS = 16


def _invmae_body(p_ref, t_ref, out_ref, vacc_ref, cacc_ref):
    i = pl.program_id(0)

    @pl.when(i == 0)
    def _init():
        vacc_ref[...] = jnp.zeros_like(vacc_ref)
        cacc_ref[...] = jnp.zeros_like(cacc_ref)

    def body(k, carry):
        aerr, acnt = carry
        sl = pl.ds(k * _CHUNK, _CHUNK)
        p = p_ref[sl, :]
        t = t_ref[sl, :]
        mask = t > 0.0
        err = jnp.where(mask, jnp.abs(p - t) / (p * t), 0.0)
        cnt = jnp.where(mask, 1.0, 0.0)
        aerr += jnp.sum(err.reshape(_CHUNK // 8, 8, 512), axis=0)
        acnt += jnp.sum(cnt.reshape(_CHUNK // 8, 8, 512), axis=0)
        return aerr, acnt

    zero = jnp.zeros((8, 512), jnp.float32)
    aerr, acnt = jax.lax.fori_loop(
        0, _ROWS // _CHUNK, body, (zero, zero), unroll=4
    )
    vacc_ref[...] += aerr
    cacc_ref[...] += acnt

    @pl.when(i == pl.num_programs(0) - 1)
    def _fin():
        out_ref[0] = jnp.sum(vacc_ref[...])
        out_ref[1] = jnp.sum(cacc_ref[...])


def _tc_partial(p, t, tc_rows):
    return pl.pallas_call(
        _invmae_body,
        grid=(tc_rows // _ROWS,),
        in_specs=[
            pl.BlockSpec((_ROWS, 512), lambda i: (i, 0)),
            pl.BlockSpec((_ROWS, 512), lambda i: (i, 0)),
        ],
        out_specs=pl.BlockSpec(memory_space=pltpu.SMEM),
        out_shape=jax.ShapeDtypeStruct((2,), jnp.float32),
        scratch_shapes=[
            pltpu.VMEM((8, 512), jnp.float32),
            pltpu.VMEM((8, 512), jnp.float32),
        ],
    )(p, t)


def _sc_make(sc_rows, sc_base_row):
    rpw = sc_rows // _NW
    n_chunks = rpw // _SC_CROWS
    mesh = plsc.VectorSubcoreMesh(core_axis_name="c", subcore_axis_name="s")

    @functools.partial(
        pl.kernel,
        mesh=mesh,
        out_type=jax.ShapeDtypeStruct((_NW, 2, _L), jnp.float32),
        scratch_types=[
            pltpu.VMEM((_SC_CROWS, 512), jnp.float32),
            pltpu.VMEM((_SC_CROWS, 512), jnp.float32),
            pltpu.VMEM((2, _L), jnp.float32),
        ],
        compiler_params=pltpu.CompilerParams(use_tc_tiling_on_sc=True),
    )
    def sc_kernel(p_hbm, t_hbm, out_hbm, p_v, t_v, res_v):
        wid = lax.axis_index("s") * _NC + lax.axis_index("c")
        base = sc_base_row + wid * rpw

        def chunk_body(g, carry):
            aerr, acnt = carry
            off = base + g * _SC_CROWS
            pltpu.sync_copy(p_hbm.at[pl.ds(off, _SC_CROWS), :], p_v)
            pltpu.sync_copy(t_hbm.at[pl.ds(off, _SC_CROWS), :], t_v)

            def inner(j, c2):
                a2, n2 = c2
                r = j // 32
                sl = pl.ds((j % 32) * _L, _L)
                p = p_v[r, sl]
                t = t_v[r, sl]
                mask = t > 0.0
                err = jnp.where(mask, jnp.abs(p - t) / (p * t), 0.0)
                one = jnp.where(mask, 1.0, 0.0)
                return a2 + err, n2 + one

            return lax.fori_loop(
                0, _SC_CROWS * 32, inner, (aerr, acnt), unroll=8
            )

        zero = jnp.zeros((_L,), jnp.float32)
        aerr, acnt = lax.fori_loop(0, n_chunks, chunk_body, (zero, zero))
        res_v[0, :] = aerr
        res_v[1, :] = acnt
        pltpu.sync_copy(res_v, out_hbm.at[wid])

    return sc_kernel


def kernel(pred, target):
    n = pred.size
    rows = n // 512
    tc_rows = rows - _SC_ROWS
    p2 = pred.reshape(rows, 512)
    t2 = target.reshape(rows, 512)

    tc_out = _tc_partial(p2, t2, tc_rows)

    sc_parts = _sc_make(_SC_ROWS, tc_rows)(p2, t2)

    s = tc_out[0] + jnp.sum(sc_parts[:, 0, :])
    c = tc_out[1] + jnp.sum(sc_parts[:, 1, :])
    loss = s / jnp.maximum(c, 1.0)
    return jnp.where(c < 10.0, jnp.float32(-1.0), loss)


# speedup vs baseline: 3.8629x; 1.4329x over previous
import jax
import jax.numpy as jnp
from jax.experimental import pallas as pl
from jax.experimental.pallas import tpu as pltpu

_ROWS = 4096
_CHUNK = 16


def _invmae_body(p_ref, t_ref, out_ref, vacc_ref, cacc_ref):
    i = pl.program_id(0)

    @pl.when(i == 0)
    def _init():
        vacc_ref[...] = jnp.zeros_like(vacc_ref)
        cacc_ref[...] = jnp.zeros_like(cacc_ref)

    def body(k, carry):
        aerr, acnt = carry
        sl = pl.ds(k * _CHUNK, _CHUNK)
        p = p_ref[sl, :]
        t = t_ref[sl, :]
        mask = t > 0.0
        err = jnp.where(mask, jnp.abs(p - t) / (p * t), 0.0)
        cnt = jnp.where(mask, 1.0, 0.0)
        aerr += jnp.sum(err.reshape(_CHUNK // 8, 8, 512), axis=0)
        acnt += jnp.sum(cnt.reshape(_CHUNK // 8, 8, 512), axis=0)
        return aerr, acnt

    zero = jnp.zeros((8, 512), jnp.float32)
    aerr, acnt = jax.lax.fori_loop(0, _ROWS // _CHUNK, body, (zero, zero), unroll=4)
    vacc_ref[...] += aerr
    cacc_ref[...] += acnt

    @pl.when(i == pl.num_programs(0) - 1)
    def _fin():
        s = jnp.sum(vacc_ref[...])
        c = jnp.sum(cacc_ref[...])
        loss = s / jnp.maximum(c, 1.0)
        out_ref[0] = jnp.where(c < 10.0, jnp.float32(-1.0), loss)


def kernel(pred, target):
    n = pred.size
    p = pred.reshape(n // 512, 512)
    t = target.reshape(n // 512, 512)
    grid = n // 512 // _ROWS
    out = pl.pallas_call(
        _invmae_body,
        grid=(grid,),
        in_specs=[
            pl.BlockSpec((_ROWS, 512), lambda i: (i, 0)),
            pl.BlockSpec((_ROWS, 512), lambda i: (i, 0)),
        ],
        out_specs=pl.BlockSpec(memory_space=pltpu.SMEM),
        out_shape=jax.ShapeDtypeStruct((1,), jnp.float32),
        scratch_shapes=[
            pltpu.VMEM((8, 512), jnp.float32),
            pltpu.VMEM((8, 512), jnp.float32),
        ],
    )(p, t)
    return out[0]
